# baseline (speedup 1.0000x reference)
import functools

import jax
import jax.numpy as jnp
from jax import lax
from jax.experimental import pallas as pl
from jax.experimental.pallas import tpu as pltpu
from jax.experimental.pallas import tpu_sc as plsc


def _build_gather(batch, seq, d):
    assert seq == 50 and d == 1024
    ng = 7
    npb = ng * 64
    info = plsc.get_sparse_core_info()
    nc, ns = info.num_cores, info.num_subcores
    nw = nc * ns
    per_w = batch // nw
    assert per_w % 2 == 0

    mesh = plsc.VectorSubcoreMesh(core_axis_name="c", subcore_axis_name="s")

    @functools.partial(
        pl.kernel,
        mesh=mesh,
        out_type=jax.ShapeDtypeStruct((batch, seq, d), jnp.float32),
        scratch_types=[
            pltpu.VMEM((npb,), jnp.int32),
            pltpu.VMEM((npb,), jnp.int32),
            pltpu.VMEM((56, d), jnp.float32),
            pltpu.VMEM((56, d), jnp.float32),
            pltpu.SemaphoreType.DMA,
            pltpu.SemaphoreType.DMA,
            pltpu.SemaphoreType.DMA,
            pltpu.SemaphoreType.DMA,
            pltpu.SemaphoreType.DMA,
            pltpu.SemaphoreType.DMA,
        ],
        compiler_params=pltpu.CompilerParams(use_tc_tiling_on_sc=True),
    )
    def gather_kernel(idx2_hbm, table_hbm, out_hbm, idx_a, idx_b, buf_a,
                      buf_b, isem_a, isem_b, gsem_a, gsem_b, osem_a, osem_b):
        wid = lax.axis_index("s") * nc + lax.axis_index("c")
        b0 = wid * per_w

        bufs = ((idx_a, buf_a, isem_a, gsem_a, osem_a),
                (idx_b, buf_b, isem_b, gsem_b, osem_b))

        def idx_src(i):
            return idx2_hbm.at[pl.ds((b0 + i) * npb, npb)]

        def start_idx(i, idxv, isem):
            pltpu.async_copy(idx_src(i), idxv, isem)

        def wait_idx(i, idxv, isem):
            pltpu.make_async_copy(idx_src(i), idxv, isem).wait()

        def gathers(idxv, buf, gsem):
            for t in range(ng):
                for c in range(8):
                    src = table_hbm.at[idxv.at[pl.ds(t * 64 + 8 * c, 8)]]
                    dst = buf.at[pl.ds(8 * t, 8), pl.ds(128 * c, 128)]
                    pltpu.async_copy(src, dst, gsem)

        def drain_gathers(buf, gsem):
            for t in range(ng):
                pltpu.make_async_copy(out_hbm.at[b0, pl.ds(0, 8)],
                                      buf.at[pl.ds(0, 8)], gsem).wait()

        def start_writes(i, buf, osem):
            b = b0 + i
            for t in range(6):
                pltpu.async_copy(buf.at[pl.ds(8 * t, 8)],
                                 out_hbm.at[b, pl.ds(8 * t, 8)], osem)
            pltpu.async_copy(buf.at[pl.ds(48, 2)],
                             out_hbm.at[b, pl.ds(48, 2)], osem)

        def wait_writes(buf, osem):
            for t in range(6):
                pltpu.make_async_copy(buf.at[pl.ds(0, 8)],
                                      out_hbm.at[b0, pl.ds(0, 8)],
                                      osem).wait()
            pltpu.make_async_copy(buf.at[pl.ds(48, 2)],
                                  out_hbm.at[b0, pl.ds(48, 2)], osem).wait()

        for b2, (idxv, buf, isem, gsem, osem) in enumerate(bufs):
            start_idx(b2, idxv, isem)
        for b2, (idxv, buf, isem, gsem, osem) in enumerate(bufs):
            wait_idx(b2, idxv, isem)
            gathers(idxv, buf, gsem)
            drain_gathers(buf, gsem)
            start_writes(b2, buf, osem)
            start_idx(b2 + 2, idxv, isem)

        def body(p, carry):
            for b2, (idxv, buf, isem, gsem, osem) in enumerate(bufs):
                i = 2 * p + b2
                wait_writes(buf, osem)
                wait_idx(i, idxv, isem)
                gathers(idxv, buf, gsem)
                drain_gathers(buf, gsem)
                start_writes(i, buf, osem)
                start_idx(i + 2, idxv, isem)
            return carry

        lax.fori_loop(1, per_w // 2, body, 0)

        for b2, (idxv, buf, isem, gsem, osem) in enumerate(bufs):
            wait_writes(buf, osem)
            wait_idx(0, idxv, isem)

    return gather_kernel


def kernel(indices, bank_embedding_weight):
    b, s = indices.shape
    v, d = bank_embedding_weight.shape
    idx32 = indices.astype(jnp.int32)
    idx_pad = jnp.pad(idx32, ((0, 0), (0, 56 - s)))
    grouped = idx_pad.reshape(b, 7, 8)
    idx2 = grouped[:, :, None, :] * 8 + jnp.arange(8)[None, None, :, None]
    idx2_flat = jnp.pad(idx2.reshape(b * 7 * 64), (0, 2 * 7 * 64))
    table_v = bank_embedding_weight.reshape(v * 8, d // 8)
    return _build_gather(b, s, d)(idx2_flat, table_v)
